# baseline (speedup 1.0000x reference)
import functools

import jax
import jax.numpy as jnp
from jax import lax
from jax.experimental import pallas as pl
from jax.experimental.pallas import tpu as pltpu
from jax.experimental.pallas import tpu_sc as plsc

NC = 2
NS = 16
LN = 512
G = 6


def _node_pad(n):
    return ((n + 1 + 127) // 128) * 128


def _edge_rows_pad(e):
    rows = (e + LN - 1) // LN
    per = NC * NS * G
    return ((rows + per - 1) // per) * per


def _deg_body(np_, rp, dst1d, zeros_hbm, ones_hbm, out, dbuf, ones_v, zbuf,
              acc, isem, ssem):
    c = lax.axis_index("c")
    s = lax.axis_index("s")
    tile_rows = np_ // NS
    per_worker = rp // (NC * NS)
    n_macro = per_worker // G

    pltpu.sync_copy(zeros_hbm, zbuf)
    pltpu.sync_copy(ones_hbm, ones_v)
    pltpu.sync_copy(zbuf, acc.at[pl.ds(s * tile_rows, tile_rows)])
    plsc.subcore_barrier()

    w = c * NS + s
    row0 = w * per_worker

    def macro(m, carry):
        base = (row0 + m * G) * LN
        pltpu.async_copy(dst1d.at[pl.ds(base, G * LN)], dbuf, isem).wait()
        descs = []
        for j in range(G):
            descs.append(
                pltpu.async_copy(ones_v,
                                 acc.at[dbuf.at[pl.ds(j * LN, LN)]], ssem,
                                 add=True))
        for d in descs:
            d.wait()
        return carry

    lax.fori_loop(0, n_macro, macro, 0)
    plsc.subcore_barrier()
    pltpu.sync_copy(acc.at[pl.ds(s * tile_rows, tile_rows)], zbuf)
    pltpu.sync_copy(zbuf, out.at[pl.ds(c * np_ + s * tile_rows, tile_rows)])


def _make_deg_call(np_, rp):
    tile_rows = np_ // NS
    mesh = plsc.VectorSubcoreMesh(core_axis_name="c", subcore_axis_name="s")
    return pl.kernel(
        functools.partial(_deg_body, np_, rp),
        out_type=jax.ShapeDtypeStruct((NC * np_,), jnp.float32),
        mesh=mesh,
        scratch_types=[
            pltpu.VMEM((G * LN,), jnp.int32),
            pltpu.VMEM((LN,), jnp.float32),
            pltpu.VMEM((tile_rows,), jnp.float32),
            pltpu.VMEM_SHARED((np_,), jnp.float32),
            pltpu.SemaphoreType.DMA,
            pltpu.SemaphoreType.DMA,
        ],
    )


def _edge_body(np_, rp, vs, src1d, dst1d, zeros_hbm, out, sbuf, dbuf, si0,
               si1, si2, di0, di1, di2, r0, r1, r2, zbuf, acc, isem, gsem,
               ssem):
    c = lax.axis_index("c")
    s = lax.axis_index("s")
    tile_words = (np_ // NS) * 4
    hw = tile_words // 2
    per_worker = rp // (NC * NS)
    n_macro = per_worker // G

    pltpu.sync_copy(zeros_hbm, zbuf)
    for k in range(2):
        pltpu.sync_copy(zbuf, acc.at[pl.ds(s * tile_words + k * hw, hw)])
    plsc.subcore_barrier()

    w = c * NS + s
    row0 = w * per_worker

    def macro(m, carry):
        base = (row0 + m * G) * LN
        dcp = pltpu.async_copy(src1d.at[pl.ds(base, G * LN)], sbuf, isem)
        pltpu.async_copy(dst1d.at[pl.ds(base, G * LN)], dbuf, isem).wait()
        dcp.wait()

        def build(v, carry2):
            o = v * 16
            ws = sbuf[pl.ds(o, 16)] * 4
            si0[pl.ds(o, 16)] = ws
            si1[pl.ds(o, 16)] = ws + 1
            si2[pl.ds(o, 16)] = ws + 2
            wd = dbuf[pl.ds(o, 16)] * 4
            di0[pl.ds(o, 16)] = wd
            di1[pl.ds(o, 16)] = wd + 1
            di2[pl.ds(o, 16)] = wd + 2
            return carry2

        lax.fori_loop(0, G * LN // 16, build, 0)

        gds = []
        for j in range(G):
            sl = pl.ds(j * LN, LN)
            gds.append(pltpu.async_copy(vs.at[si0.at[sl]], r0.at[sl], gsem))
            gds.append(pltpu.async_copy(vs.at[si1.at[sl]], r1.at[sl], gsem))
            gds.append(pltpu.async_copy(vs.at[si2.at[sl]], r2.at[sl], gsem))
        for d in gds:
            d.wait()
        sds = []
        for j in range(G):
            sl = pl.ds(j * LN, LN)
            sds.append(
                pltpu.async_copy(r0.at[sl], acc.at[di0.at[sl]], ssem,
                                 add=True))
            sds.append(
                pltpu.async_copy(r1.at[sl], acc.at[di1.at[sl]], ssem,
                                 add=True))
            sds.append(
                pltpu.async_copy(r2.at[sl], acc.at[di2.at[sl]], ssem,
                                 add=True))
        for d in sds:
            d.wait()
        return carry

    lax.fori_loop(0, n_macro, macro, 0)
    plsc.subcore_barrier()
    for k in range(2):
        pltpu.sync_copy(acc.at[pl.ds(s * tile_words + k * hw, hw)], zbuf)
        pltpu.sync_copy(
            zbuf, out.at[pl.ds(c * np_ * 4 + s * tile_words + k * hw, hw)])


def _make_edge_call(np_, rp):
    tile_words = (np_ // NS) * 4
    mesh = plsc.VectorSubcoreMesh(core_axis_name="c", subcore_axis_name="s")
    ibuf = pltpu.VMEM((G * LN,), jnp.int32)
    fbuf = pltpu.VMEM((G * LN,), jnp.float32)
    return pl.kernel(
        functools.partial(_edge_body, np_, rp),
        out_type=jax.ShapeDtypeStruct((NC * np_ * 4,), jnp.float32),
        mesh=mesh,
        scratch_types=[
            ibuf, ibuf, ibuf, ibuf, ibuf, ibuf, ibuf, ibuf,
            fbuf, fbuf, fbuf,
            pltpu.VMEM((tile_words // 2,), jnp.float32),
            pltpu.VMEM_SHARED((np_ * 4,), jnp.float32),
            pltpu.SemaphoreType.DMA,
            pltpu.SemaphoreType.DMA,
            pltpu.SemaphoreType.DMA,
        ],
    )


def _stage_b_body(degp, xp, vs1, dis, inv):
    deg = degp[0, :][:, None] + degp[1, :][:, None] + 1.0
    d = lax.rsqrt(deg)
    iv = 1.0 / deg
    dis[...] = d
    inv[...] = iv
    vs1[...] = xp[...] * d


def _stage_d_body(aggp, xp, dis, inv, w1, b1, w2, vs2, self2):
    a = aggp[0] + aggp[1]
    d = dis[...]
    iv = inv[...]
    pre = d * a[:, :3] + iv * xp[:, :3]
    h = jnp.maximum(
        lax.dot_general(pre, w1[...], (((1,), (0,)), ((), ())),
                        preferred_element_type=jnp.float32) + b1[...][None, :],
        0.0)
    y = lax.dot_general(h, w2[...], (((1,), (0,)), ((), ())),
                        preferred_element_type=jnp.float32)
    vs2[...] = jnp.concatenate(
        [y * d, jnp.zeros((y.shape[0], 1), jnp.float32)], axis=1)
    self2[...] = y * iv


def _stage_f_body(aggp, self2, dis, b2, out):
    a = aggp[0] + aggp[1]
    out[...] = dis[...] * a[:, :3] + self2[...] + b2[...][None, :]


def _dense_calls(np_):
    n128 = np_ // 128
    grid = max(d for d in range(1, 65) if n128 % d == 0)
    rb = np_ // grid
    full = lambda shape: pl.BlockSpec(shape, lambda i: tuple(0 for _ in shape))
    row1 = pl.BlockSpec((rb, 1), lambda i: (i, 0))
    row4 = pl.BlockSpec((rb, 4), lambda i: (i, 0))
    row3 = pl.BlockSpec((rb, 3), lambda i: (i, 0))
    agg = pl.BlockSpec((2, rb, 4), lambda i: (0, i, 0))

    stage_b = pl.pallas_call(
        _stage_b_body,
        grid=(grid,),
        in_specs=[pl.BlockSpec((2, rb), lambda i: (0, i)), row4],
        out_specs=[row4, row1, row1],
        out_shape=[
            jax.ShapeDtypeStruct((np_, 4), jnp.float32),
            jax.ShapeDtypeStruct((np_, 1), jnp.float32),
            jax.ShapeDtypeStruct((np_, 1), jnp.float32),
        ],
    )
    stage_d = pl.pallas_call(
        _stage_d_body,
        grid=(grid,),
        in_specs=[agg, row4, row1, row1, full((3, 16)), full((16,)),
                  full((16, 3))],
        out_specs=[row4, row3],
        out_shape=[
            jax.ShapeDtypeStruct((np_, 4), jnp.float32),
            jax.ShapeDtypeStruct((np_, 3), jnp.float32),
        ],
    )
    stage_f = pl.pallas_call(
        _stage_f_body,
        grid=(grid,),
        in_specs=[agg, row3, row1, full((3,))],
        out_specs=row3,
        out_shape=jax.ShapeDtypeStruct((np_, 3), jnp.float32),
    )
    return stage_b, stage_d, stage_f


def kernel(x, edge_index, W1, b1, W2, b2):
    n = x.shape[0]
    e = edge_index.shape[1]
    np_ = _node_pad(n)
    rp = _edge_rows_pad(e)
    tile_rows = np_ // NS

    src = edge_index[0]
    dst = edge_index[1]
    pad = rp * LN - e
    src1d = jnp.concatenate([src, jnp.zeros((pad,), jnp.int32)])
    dst1d = jnp.concatenate([dst, jnp.full((pad,), n, jnp.int32)])
    xp = jnp.pad(x, ((0, np_ - n), (0, 1)))

    zeros1 = jnp.zeros((tile_rows,), jnp.float32)
    zeros2 = jnp.zeros((tile_rows * 2,), jnp.float32)
    ones_l = jnp.ones((LN,), jnp.float32)

    deg_call = _make_deg_call(np_, rp)
    edge_call = _make_edge_call(np_, rp)
    stage_b, stage_d, stage_f = _dense_calls(np_)

    degp = deg_call(dst1d, zeros1, ones_l).reshape(NC, np_)
    vs1, dis, inv = stage_b(degp, xp)
    aggp1 = edge_call(vs1.reshape(-1), src1d, dst1d,
                      zeros2).reshape(NC, np_, 4)
    vs2, self2 = stage_d(aggp1, xp, dis, inv, W1, b1, W2)
    aggp2 = edge_call(vs2.reshape(-1), src1d, dst1d,
                      zeros2).reshape(NC, np_, 4)
    outp = stage_f(aggp2, self2, dis, b2)
    return outp[:n]


# speedup vs baseline: 1.7059x; 1.7059x over previous
import functools

import jax
import jax.numpy as jnp
from jax import lax
from jax.experimental import pallas as pl
from jax.experimental.pallas import tpu as pltpu
from jax.experimental.pallas import tpu_sc as plsc

NC = 2
NS = 16
LN = 128
G = 23


def _node_pad(n):
    return ((n + 1 + 127) // 128) * 128


def _edge_rows_pad(e):
    rows = (e + LN - 1) // LN
    per = NC * NS * G
    return ((rows + per - 1) // per) * per


def _deg_body(np_, rp, dst1d, zeros_hbm, ones_hbm, out, dbuf, ones_v, zbuf,
              acc, isem, ssem):
    c = lax.axis_index("c")
    s = lax.axis_index("s")
    tile_rows = np_ // NS
    per_worker = rp // (NC * NS)
    n_macro = per_worker // G

    pltpu.sync_copy(zeros_hbm, zbuf)
    pltpu.sync_copy(ones_hbm, ones_v)
    pltpu.sync_copy(zbuf, acc.at[pl.ds(s * tile_rows, tile_rows)])
    plsc.subcore_barrier()

    w = c * NS + s
    row0 = w * per_worker

    def macro(m, carry):
        base = (row0 + m * G) * LN
        pltpu.async_copy(dst1d.at[pl.ds(base, G * LN)], dbuf, isem).wait()
        descs = []
        for j in range(G):
            descs.append(
                pltpu.async_copy(ones_v,
                                 acc.at[dbuf.at[pl.ds(j * LN, LN)]], ssem,
                                 add=True))
        for d in descs:
            d.wait()
        return carry

    lax.fori_loop(0, n_macro, macro, 0)
    plsc.subcore_barrier()
    pltpu.sync_copy(acc.at[pl.ds(s * tile_rows, tile_rows)], zbuf)
    pltpu.sync_copy(zbuf, out.at[pl.ds(c * np_ + s * tile_rows, tile_rows)])


def _make_deg_call(np_, rp):
    tile_rows = np_ // NS
    mesh = plsc.VectorSubcoreMesh(core_axis_name="c", subcore_axis_name="s")
    return pl.kernel(
        functools.partial(_deg_body, np_, rp),
        out_type=jax.ShapeDtypeStruct((NC * np_,), jnp.float32),
        mesh=mesh,
        scratch_types=[
            pltpu.VMEM((G * LN,), jnp.int32),
            pltpu.VMEM((LN,), jnp.float32),
            pltpu.VMEM((tile_rows,), jnp.float32),
            pltpu.VMEM_SHARED((np_,), jnp.float32),
            pltpu.SemaphoreType.DMA,
            pltpu.SemaphoreType.DMA,
        ],
    )


def _edge_body(np_, rp, vs, src1d, dst1d, zeros_hbm, out, sbuf, dbuf, si0,
               si1, si2, di0, di1, di2, r0, r1, r2, zbuf, acc, isem, gsem,
               ssem):
    c = lax.axis_index("c")
    s = lax.axis_index("s")
    tile_words = (np_ // NS) * 4
    hw = tile_words // 2
    per_worker = rp // (NC * NS)
    n_macro = per_worker // G

    pltpu.sync_copy(zeros_hbm, zbuf)
    for k in range(2):
        pltpu.sync_copy(zbuf, acc.at[pl.ds(s * tile_words + k * hw, hw)])
    plsc.subcore_barrier()

    w = c * NS + s
    row0 = w * per_worker

    def macro(m, carry):
        base = (row0 + m * G) * LN
        dcp = pltpu.async_copy(src1d.at[pl.ds(base, G * LN)], sbuf, isem)
        pltpu.async_copy(dst1d.at[pl.ds(base, G * LN)], dbuf, isem).wait()
        dcp.wait()

        def build(v, carry2):
            o = v * 16
            ws = sbuf[pl.ds(o, 16)] * 4
            si0[pl.ds(o, 16)] = ws
            si1[pl.ds(o, 16)] = ws + 1
            si2[pl.ds(o, 16)] = ws + 2
            wd = dbuf[pl.ds(o, 16)] * 4
            di0[pl.ds(o, 16)] = wd
            di1[pl.ds(o, 16)] = wd + 1
            di2[pl.ds(o, 16)] = wd + 2
            return carry2

        lax.fori_loop(0, G * LN // 16, build, 0)

        gds = []
        for j in range(G):
            sl = pl.ds(j * LN, LN)
            gds.append(pltpu.async_copy(vs.at[si0.at[sl]], r0.at[sl], gsem))
            gds.append(pltpu.async_copy(vs.at[si1.at[sl]], r1.at[sl], gsem))
            gds.append(pltpu.async_copy(vs.at[si2.at[sl]], r2.at[sl], gsem))
        sds = []
        for j in range(G):
            sl = pl.ds(j * LN, LN)
            gds[3 * j].wait()
            gds[3 * j + 1].wait()
            gds[3 * j + 2].wait()
            sds.append(
                pltpu.async_copy(r0.at[sl], acc.at[di0.at[sl]], ssem,
                                 add=True))
            sds.append(
                pltpu.async_copy(r1.at[sl], acc.at[di1.at[sl]], ssem,
                                 add=True))
            sds.append(
                pltpu.async_copy(r2.at[sl], acc.at[di2.at[sl]], ssem,
                                 add=True))
        for d in sds:
            d.wait()
        return carry

    lax.fori_loop(0, n_macro, macro, 0)
    plsc.subcore_barrier()
    for k in range(2):
        pltpu.sync_copy(acc.at[pl.ds(s * tile_words + k * hw, hw)], zbuf)
        pltpu.sync_copy(
            zbuf, out.at[pl.ds(c * np_ * 4 + s * tile_words + k * hw, hw)])


def _make_edge_call(np_, rp):
    tile_words = (np_ // NS) * 4
    mesh = plsc.VectorSubcoreMesh(core_axis_name="c", subcore_axis_name="s")
    ibuf = pltpu.VMEM((G * LN,), jnp.int32)
    fbuf = pltpu.VMEM((G * LN,), jnp.float32)
    return pl.kernel(
        functools.partial(_edge_body, np_, rp),
        out_type=jax.ShapeDtypeStruct((NC * np_ * 4,), jnp.float32),
        mesh=mesh,
        scratch_types=[
            ibuf, ibuf, ibuf, ibuf, ibuf, ibuf, ibuf, ibuf,
            fbuf, fbuf, fbuf,
            pltpu.VMEM((tile_words // 2,), jnp.float32),
            pltpu.VMEM_SHARED((np_ * 4,), jnp.float32),
            pltpu.SemaphoreType.DMA,
            pltpu.SemaphoreType.DMA,
            pltpu.SemaphoreType.DMA,
        ],
    )


def _stage_b_body(degp, xp, vs1, dis, inv):
    deg = degp[0, :][:, None] + degp[1, :][:, None] + 1.0
    d = lax.rsqrt(deg)
    iv = 1.0 / deg
    dis[...] = d
    inv[...] = iv
    vs1[...] = xp[...] * d


def _stage_d_body(aggp, xp, dis, inv, w1, b1, w2, vs2, self2):
    a = aggp[0] + aggp[1]
    d = dis[...]
    iv = inv[...]
    pre = d * a[:, :3] + iv * xp[:, :3]
    h = jnp.maximum(
        lax.dot_general(pre, w1[...], (((1,), (0,)), ((), ())),
                        preferred_element_type=jnp.float32) + b1[...][None, :],
        0.0)
    y = lax.dot_general(h, w2[...], (((1,), (0,)), ((), ())),
                        preferred_element_type=jnp.float32)
    vs2[...] = jnp.concatenate(
        [y * d, jnp.zeros((y.shape[0], 1), jnp.float32)], axis=1)
    self2[...] = y * iv


def _stage_f_body(aggp, self2, dis, b2, out):
    a = aggp[0] + aggp[1]
    out[...] = dis[...] * a[:, :3] + self2[...] + b2[...][None, :]


def _dense_calls(np_):
    n128 = np_ // 128
    grid = max(d for d in range(1, 65) if n128 % d == 0)
    rb = np_ // grid
    full = lambda shape: pl.BlockSpec(shape, lambda i: tuple(0 for _ in shape))
    row1 = pl.BlockSpec((rb, 1), lambda i: (i, 0))
    row4 = pl.BlockSpec((rb, 4), lambda i: (i, 0))
    row3 = pl.BlockSpec((rb, 3), lambda i: (i, 0))
    agg = pl.BlockSpec((2, rb, 4), lambda i: (0, i, 0))

    stage_b = pl.pallas_call(
        _stage_b_body,
        grid=(grid,),
        in_specs=[pl.BlockSpec((2, rb), lambda i: (0, i)), row4],
        out_specs=[row4, row1, row1],
        out_shape=[
            jax.ShapeDtypeStruct((np_, 4), jnp.float32),
            jax.ShapeDtypeStruct((np_, 1), jnp.float32),
            jax.ShapeDtypeStruct((np_, 1), jnp.float32),
        ],
    )
    stage_d = pl.pallas_call(
        _stage_d_body,
        grid=(grid,),
        in_specs=[agg, row4, row1, row1, full((3, 16)), full((16,)),
                  full((16, 3))],
        out_specs=[row4, row3],
        out_shape=[
            jax.ShapeDtypeStruct((np_, 4), jnp.float32),
            jax.ShapeDtypeStruct((np_, 3), jnp.float32),
        ],
    )
    stage_f = pl.pallas_call(
        _stage_f_body,
        grid=(grid,),
        in_specs=[agg, row3, row1, full((3,))],
        out_specs=row3,
        out_shape=jax.ShapeDtypeStruct((np_, 3), jnp.float32),
    )
    return stage_b, stage_d, stage_f


def kernel(x, edge_index, W1, b1, W2, b2):
    n = x.shape[0]
    e = edge_index.shape[1]
    np_ = _node_pad(n)
    rp = _edge_rows_pad(e)
    tile_rows = np_ // NS

    src = edge_index[0]
    dst = edge_index[1]
    pad = rp * LN - e
    src1d = jnp.concatenate([src, jnp.zeros((pad,), jnp.int32)])
    dst1d = jnp.concatenate([dst, jnp.full((pad,), n, jnp.int32)])
    xp = jnp.pad(x, ((0, np_ - n), (0, 1)))

    zeros1 = jnp.zeros((tile_rows,), jnp.float32)
    zeros2 = jnp.zeros((tile_rows * 2,), jnp.float32)
    ones_l = jnp.ones((LN,), jnp.float32)

    deg_call = _make_deg_call(np_, rp)
    edge_call = _make_edge_call(np_, rp)
    stage_b, stage_d, stage_f = _dense_calls(np_)

    degp = deg_call(dst1d, zeros1, ones_l).reshape(NC, np_)
    vs1, dis, inv = stage_b(degp, xp)
    aggp1 = edge_call(vs1.reshape(-1), src1d, dst1d,
                      zeros2).reshape(NC, np_, 4)
    vs2, self2 = stage_d(aggp1, xp, dis, inv, W1, b1, W2)
    aggp2 = edge_call(vs2.reshape(-1), src1d, dst1d,
                      zeros2).reshape(NC, np_, 4)
    outp = stage_f(aggp2, self2, dis, b2)
    return outp[:n]


# speedup vs baseline: 1.7561x; 1.0294x over previous
import functools

import jax
import jax.numpy as jnp
from jax import lax
from jax.experimental import pallas as pl
from jax.experimental.pallas import tpu as pltpu
from jax.experimental.pallas import tpu_sc as plsc

NC = 2
NS = 16
LN = 128
G = 46


def _node_pad(n):
    return ((n + 1 + 127) // 128) * 128


def _edge_rows_pad(e):
    rows = (e + LN - 1) // LN
    per = NC * NS * G
    return ((rows + per - 1) // per) * per


def _deg_body(np_, rp, dst1d, zeros_hbm, ones_hbm, out, dbuf, ones_v, zbuf,
              acc, isem, ssem):
    c = lax.axis_index("c")
    s = lax.axis_index("s")
    tile_rows = np_ // NS
    per_worker = rp // (NC * NS)
    n_macro = per_worker // G

    pltpu.sync_copy(zeros_hbm, zbuf)
    pltpu.sync_copy(ones_hbm, ones_v)
    pltpu.sync_copy(zbuf, acc.at[pl.ds(s * tile_rows, tile_rows)])
    plsc.subcore_barrier()

    w = c * NS + s
    row0 = w * per_worker

    def macro(m, carry):
        base = (row0 + m * G) * LN
        pltpu.async_copy(dst1d.at[pl.ds(base, G * LN)], dbuf, isem).wait()
        descs = []
        for j in range(G):
            descs.append(
                pltpu.async_copy(ones_v,
                                 acc.at[dbuf.at[pl.ds(j * LN, LN)]], ssem,
                                 add=True))
        for d in descs:
            d.wait()
        return carry

    lax.fori_loop(0, n_macro, macro, 0)
    plsc.subcore_barrier()
    pltpu.sync_copy(acc.at[pl.ds(s * tile_rows, tile_rows)], zbuf)
    pltpu.sync_copy(zbuf, out.at[pl.ds(c * np_ + s * tile_rows, tile_rows)])


def _make_deg_call(np_, rp):
    tile_rows = np_ // NS
    mesh = plsc.VectorSubcoreMesh(core_axis_name="c", subcore_axis_name="s")
    return pl.kernel(
        functools.partial(_deg_body, np_, rp),
        out_type=jax.ShapeDtypeStruct((NC * np_,), jnp.float32),
        mesh=mesh,
        scratch_types=[
            pltpu.VMEM((G * LN,), jnp.int32),
            pltpu.VMEM((LN,), jnp.float32),
            pltpu.VMEM((tile_rows,), jnp.float32),
            pltpu.VMEM_SHARED((np_,), jnp.float32),
            pltpu.SemaphoreType.DMA,
            pltpu.SemaphoreType.DMA,
        ],
    )


def _edge_body(np_, rp, vs, src1d, dst1d, zeros_hbm, out, sbuf, dbuf, si0,
               si1, si2, di0, di1, di2, r0, r1, r2, zbuf, acc, isem, gsem,
               ssem):
    c = lax.axis_index("c")
    s = lax.axis_index("s")
    tile_words = (np_ // NS) * 4
    hw = tile_words // 2
    per_worker = rp // (NC * NS)
    n_macro = per_worker // G

    pltpu.sync_copy(zeros_hbm, zbuf)
    for k in range(2):
        pltpu.sync_copy(zbuf, acc.at[pl.ds(s * tile_words + k * hw, hw)])
    plsc.subcore_barrier()

    w = c * NS + s
    row0 = w * per_worker

    def macro(m, carry):
        base = (row0 + m * G) * LN
        dcp = pltpu.async_copy(src1d.at[pl.ds(base, G * LN)], sbuf, isem)
        pltpu.async_copy(dst1d.at[pl.ds(base, G * LN)], dbuf, isem).wait()
        dcp.wait()

        def build(v, carry2):
            o = v * 16
            ws = sbuf[pl.ds(o, 16)] * 4
            si0[pl.ds(o, 16)] = ws
            si1[pl.ds(o, 16)] = ws + 1
            si2[pl.ds(o, 16)] = ws + 2
            wd = dbuf[pl.ds(o, 16)] * 4
            di0[pl.ds(o, 16)] = wd
            di1[pl.ds(o, 16)] = wd + 1
            di2[pl.ds(o, 16)] = wd + 2
            return carry2

        lax.fori_loop(0, G * LN // 16, build, 0)

        gds = []
        for j in range(G):
            sl = pl.ds(j * LN, LN)
            gds.append(pltpu.async_copy(vs.at[si0.at[sl]], r0.at[sl], gsem))
            gds.append(pltpu.async_copy(vs.at[si1.at[sl]], r1.at[sl], gsem))
            gds.append(pltpu.async_copy(vs.at[si2.at[sl]], r2.at[sl], gsem))
        sds = []
        for j in range(G):
            sl = pl.ds(j * LN, LN)
            gds[3 * j].wait()
            gds[3 * j + 1].wait()
            gds[3 * j + 2].wait()
            sds.append(
                pltpu.async_copy(r0.at[sl], acc.at[di0.at[sl]], ssem,
                                 add=True))
            sds.append(
                pltpu.async_copy(r1.at[sl], acc.at[di1.at[sl]], ssem,
                                 add=True))
            sds.append(
                pltpu.async_copy(r2.at[sl], acc.at[di2.at[sl]], ssem,
                                 add=True))
        for d in sds:
            d.wait()
        return carry

    lax.fori_loop(0, n_macro, macro, 0)
    plsc.subcore_barrier()
    for k in range(2):
        pltpu.sync_copy(acc.at[pl.ds(s * tile_words + k * hw, hw)], zbuf)
        pltpu.sync_copy(
            zbuf, out.at[pl.ds(c * np_ * 4 + s * tile_words + k * hw, hw)])


def _make_edge_call(np_, rp):
    tile_words = (np_ // NS) * 4
    mesh = plsc.VectorSubcoreMesh(core_axis_name="c", subcore_axis_name="s")
    ibuf = pltpu.VMEM((G * LN,), jnp.int32)
    fbuf = pltpu.VMEM((G * LN,), jnp.float32)
    return pl.kernel(
        functools.partial(_edge_body, np_, rp),
        out_type=jax.ShapeDtypeStruct((NC * np_ * 4,), jnp.float32),
        mesh=mesh,
        scratch_types=[
            ibuf, ibuf, ibuf, ibuf, ibuf, ibuf, ibuf, ibuf,
            fbuf, fbuf, fbuf,
            pltpu.VMEM((tile_words // 2,), jnp.float32),
            pltpu.VMEM_SHARED((np_ * 4,), jnp.float32),
            pltpu.SemaphoreType.DMA,
            pltpu.SemaphoreType.DMA,
            pltpu.SemaphoreType.DMA,
        ],
    )


def _stage_b_body(degp, xp, vs1, dis, inv):
    deg = degp[0, :][:, None] + degp[1, :][:, None] + 1.0
    d = lax.rsqrt(deg)
    iv = 1.0 / deg
    dis[...] = d
    inv[...] = iv
    vs1[...] = xp[...] * d


def _stage_d_body(aggp, xp, dis, inv, w1, b1, w2, vs2, self2):
    a = aggp[0] + aggp[1]
    d = dis[...]
    iv = inv[...]
    pre = d * a[:, :3] + iv * xp[:, :3]
    h = jnp.maximum(
        lax.dot_general(pre, w1[...], (((1,), (0,)), ((), ())),
                        preferred_element_type=jnp.float32) + b1[...][None, :],
        0.0)
    y = lax.dot_general(h, w2[...], (((1,), (0,)), ((), ())),
                        preferred_element_type=jnp.float32)
    vs2[...] = jnp.concatenate(
        [y * d, jnp.zeros((y.shape[0], 1), jnp.float32)], axis=1)
    self2[...] = y * iv


def _stage_f_body(aggp, self2, dis, b2, out):
    a = aggp[0] + aggp[1]
    out[...] = dis[...] * a[:, :3] + self2[...] + b2[...][None, :]


def _dense_calls(np_):
    n128 = np_ // 128
    grid = max(d for d in range(1, 65) if n128 % d == 0)
    rb = np_ // grid
    full = lambda shape: pl.BlockSpec(shape, lambda i: tuple(0 for _ in shape))
    row1 = pl.BlockSpec((rb, 1), lambda i: (i, 0))
    row4 = pl.BlockSpec((rb, 4), lambda i: (i, 0))
    row3 = pl.BlockSpec((rb, 3), lambda i: (i, 0))
    agg = pl.BlockSpec((2, rb, 4), lambda i: (0, i, 0))

    stage_b = pl.pallas_call(
        _stage_b_body,
        grid=(grid,),
        in_specs=[pl.BlockSpec((2, rb), lambda i: (0, i)), row4],
        out_specs=[row4, row1, row1],
        out_shape=[
            jax.ShapeDtypeStruct((np_, 4), jnp.float32),
            jax.ShapeDtypeStruct((np_, 1), jnp.float32),
            jax.ShapeDtypeStruct((np_, 1), jnp.float32),
        ],
    )
    stage_d = pl.pallas_call(
        _stage_d_body,
        grid=(grid,),
        in_specs=[agg, row4, row1, row1, full((3, 16)), full((16,)),
                  full((16, 3))],
        out_specs=[row4, row3],
        out_shape=[
            jax.ShapeDtypeStruct((np_, 4), jnp.float32),
            jax.ShapeDtypeStruct((np_, 3), jnp.float32),
        ],
    )
    stage_f = pl.pallas_call(
        _stage_f_body,
        grid=(grid,),
        in_specs=[agg, row3, row1, full((3,))],
        out_specs=row3,
        out_shape=jax.ShapeDtypeStruct((np_, 3), jnp.float32),
    )
    return stage_b, stage_d, stage_f


def kernel(x, edge_index, W1, b1, W2, b2):
    n = x.shape[0]
    e = edge_index.shape[1]
    np_ = _node_pad(n)
    rp = _edge_rows_pad(e)
    tile_rows = np_ // NS

    src = edge_index[0]
    dst = edge_index[1]
    pad = rp * LN - e
    src1d = jnp.concatenate([src, jnp.zeros((pad,), jnp.int32)])
    dst1d = jnp.concatenate([dst, jnp.full((pad,), n, jnp.int32)])
    xp = jnp.pad(x, ((0, np_ - n), (0, 1)))

    zeros1 = jnp.zeros((tile_rows,), jnp.float32)
    zeros2 = jnp.zeros((tile_rows * 2,), jnp.float32)
    ones_l = jnp.ones((LN,), jnp.float32)

    deg_call = _make_deg_call(np_, rp)
    edge_call = _make_edge_call(np_, rp)
    stage_b, stage_d, stage_f = _dense_calls(np_)

    degp = deg_call(dst1d, zeros1, ones_l).reshape(NC, np_)
    vs1, dis, inv = stage_b(degp, xp)
    aggp1 = edge_call(vs1.reshape(-1), src1d, dst1d,
                      zeros2).reshape(NC, np_, 4)
    vs2, self2 = stage_d(aggp1, xp, dis, inv, W1, b1, W2)
    aggp2 = edge_call(vs2.reshape(-1), src1d, dst1d,
                      zeros2).reshape(NC, np_, 4)
    outp = stage_f(aggp2, self2, dis, b2)
    return outp[:n]


# speedup vs baseline: 2.5166x; 1.4330x over previous
import functools

import jax
import jax.numpy as jnp
from jax import lax
from jax.experimental import pallas as pl
from jax.experimental.pallas import tpu as pltpu
from jax.experimental.pallas import tpu_sc as plsc

NC = 2
NS = 16
LN = 128
G = 46


def _node_pad(n):
    return ((n + 1 + 127) // 128) * 128


def _edge_rows_pad(e):
    rows = (e + LN - 1) // LN
    per = NC * NS * G
    return ((rows + per - 1) // per) * per


def _deg_body(np_, rp, dst1d, zeros_hbm, ones_hbm, out, dbuf, ones_v, zbuf,
              acc, isem, ssem):
    c = lax.axis_index("c")
    s = lax.axis_index("s")
    tile_rows = np_ // NS
    per_worker = rp // (NC * NS)
    n_macro = per_worker // G

    pltpu.sync_copy(zeros_hbm, zbuf)
    pltpu.sync_copy(ones_hbm, ones_v)
    pltpu.sync_copy(zbuf, acc.at[pl.ds(s * tile_rows, tile_rows)])
    plsc.subcore_barrier()

    w = c * NS + s
    row0 = w * per_worker

    def macro(m, carry):
        base = (row0 + m * G) * LN
        pltpu.async_copy(dst1d.at[pl.ds(base, G * LN)], dbuf, isem).wait()
        descs = []
        for j in range(G):
            descs.append(
                pltpu.async_copy(ones_v,
                                 acc.at[dbuf.at[pl.ds(j * LN, LN)]], ssem,
                                 add=True))
        for d in descs:
            d.wait()
        return carry

    lax.fori_loop(0, n_macro, macro, 0)
    plsc.subcore_barrier()
    pltpu.sync_copy(acc.at[pl.ds(s * tile_rows, tile_rows)], zbuf)
    pltpu.sync_copy(zbuf, out.at[pl.ds(c * np_ + s * tile_rows, tile_rows)])


def _make_deg_call(np_, rp):
    tile_rows = np_ // NS
    mesh = plsc.VectorSubcoreMesh(core_axis_name="c", subcore_axis_name="s")
    return pl.kernel(
        functools.partial(_deg_body, np_, rp),
        out_type=jax.ShapeDtypeStruct((NC * np_,), jnp.float32),
        mesh=mesh,
        scratch_types=[
            pltpu.VMEM((G * LN,), jnp.int32),
            pltpu.VMEM((LN,), jnp.float32),
            pltpu.VMEM((tile_rows,), jnp.float32),
            pltpu.VMEM_SHARED((np_,), jnp.float32),
            pltpu.SemaphoreType.DMA,
            pltpu.SemaphoreType.DMA,
        ],
    )


def _edge_body(np_, rp, vs, src1d, dst1d, zeros_hbm, out, sbuf, dbuf, si0,
               si1, si2, di0, di1, di2, r0, r1, r2, zbuf, tbl, acc, isem,
               gsem, ssem):
    c = lax.axis_index("c")
    s = lax.axis_index("s")
    tile_words = (np_ // NS) * 4
    hw = tile_words // 2
    per_worker = rp // (NC * NS)
    n_macro = per_worker // G

    for k in range(2):
        w0 = s * tile_words + k * hw
        pltpu.sync_copy(vs.at[pl.ds(w0, hw)], zbuf)
        pltpu.sync_copy(zbuf, tbl.at[pl.ds(w0, hw)])
    pltpu.sync_copy(zeros_hbm, zbuf)
    for k in range(2):
        pltpu.sync_copy(zbuf, acc.at[pl.ds(s * tile_words + k * hw, hw)])
    plsc.subcore_barrier()

    w = c * NS + s
    row0 = w * per_worker

    def macro(m, carry):
        base = (row0 + m * G) * LN
        dcp = pltpu.async_copy(src1d.at[pl.ds(base, G * LN)], sbuf, isem)
        pltpu.async_copy(dst1d.at[pl.ds(base, G * LN)], dbuf, isem).wait()
        dcp.wait()

        def build(v, carry2):
            o = v * 16
            ws = sbuf[pl.ds(o, 16)] * 4
            si0[pl.ds(o, 16)] = ws
            si1[pl.ds(o, 16)] = ws + 1
            si2[pl.ds(o, 16)] = ws + 2
            wd = dbuf[pl.ds(o, 16)] * 4
            di0[pl.ds(o, 16)] = wd
            di1[pl.ds(o, 16)] = wd + 1
            di2[pl.ds(o, 16)] = wd + 2
            return carry2

        lax.fori_loop(0, G * LN // 16, build, 0)

        gds = []
        for j in range(G):
            sl = pl.ds(j * LN, LN)
            gds.append(pltpu.async_copy(tbl.at[si0.at[sl]], r0.at[sl], gsem))
            gds.append(pltpu.async_copy(tbl.at[si1.at[sl]], r1.at[sl], gsem))
            gds.append(pltpu.async_copy(tbl.at[si2.at[sl]], r2.at[sl], gsem))
        sds = []
        for j in range(G):
            sl = pl.ds(j * LN, LN)
            gds[3 * j].wait()
            gds[3 * j + 1].wait()
            gds[3 * j + 2].wait()
            sds.append(
                pltpu.async_copy(r0.at[sl], acc.at[di0.at[sl]], ssem,
                                 add=True))
            sds.append(
                pltpu.async_copy(r1.at[sl], acc.at[di1.at[sl]], ssem,
                                 add=True))
            sds.append(
                pltpu.async_copy(r2.at[sl], acc.at[di2.at[sl]], ssem,
                                 add=True))
        for d in sds:
            d.wait()
        return carry

    lax.fori_loop(0, n_macro, macro, 0)
    plsc.subcore_barrier()
    for k in range(2):
        pltpu.sync_copy(acc.at[pl.ds(s * tile_words + k * hw, hw)], zbuf)
        pltpu.sync_copy(
            zbuf, out.at[pl.ds(c * np_ * 4 + s * tile_words + k * hw, hw)])


def _make_edge_call(np_, rp):
    tile_words = (np_ // NS) * 4
    mesh = plsc.VectorSubcoreMesh(core_axis_name="c", subcore_axis_name="s")
    ibuf = pltpu.VMEM((G * LN,), jnp.int32)
    fbuf = pltpu.VMEM((G * LN,), jnp.float32)
    return pl.kernel(
        functools.partial(_edge_body, np_, rp),
        out_type=jax.ShapeDtypeStruct((NC * np_ * 4,), jnp.float32),
        mesh=mesh,
        scratch_types=[
            ibuf, ibuf, ibuf, ibuf, ibuf, ibuf, ibuf, ibuf,
            fbuf, fbuf, fbuf,
            pltpu.VMEM((tile_words // 2,), jnp.float32),
            pltpu.VMEM_SHARED((np_ * 4,), jnp.float32),
            pltpu.VMEM_SHARED((np_ * 4,), jnp.float32),
            pltpu.SemaphoreType.DMA,
            pltpu.SemaphoreType.DMA,
            pltpu.SemaphoreType.DMA,
        ],
    )


def _stage_b_body(degp, xp, vs1, dis, inv):
    deg = degp[0, :][:, None] + degp[1, :][:, None] + 1.0
    d = lax.rsqrt(deg)
    iv = 1.0 / deg
    dis[...] = d
    inv[...] = iv
    vs1[...] = xp[...] * d


def _stage_d_body(aggp, xp, dis, inv, w1, b1, w2, vs2, self2):
    a = aggp[0] + aggp[1]
    d = dis[...]
    iv = inv[...]
    pre = d * a[:, :3] + iv * xp[:, :3]
    h = jnp.maximum(
        lax.dot_general(pre, w1[...], (((1,), (0,)), ((), ())),
                        preferred_element_type=jnp.float32) + b1[...][None, :],
        0.0)
    y = lax.dot_general(h, w2[...], (((1,), (0,)), ((), ())),
                        preferred_element_type=jnp.float32)
    vs2[...] = jnp.concatenate(
        [y * d, jnp.zeros((y.shape[0], 1), jnp.float32)], axis=1)
    self2[...] = y * iv


def _stage_f_body(aggp, self2, dis, b2, out):
    a = aggp[0] + aggp[1]
    out[...] = dis[...] * a[:, :3] + self2[...] + b2[...][None, :]


def _dense_calls(np_):
    n128 = np_ // 128
    grid = max(d for d in range(1, 65) if n128 % d == 0)
    rb = np_ // grid
    full = lambda shape: pl.BlockSpec(shape, lambda i: tuple(0 for _ in shape))
    row1 = pl.BlockSpec((rb, 1), lambda i: (i, 0))
    row4 = pl.BlockSpec((rb, 4), lambda i: (i, 0))
    row3 = pl.BlockSpec((rb, 3), lambda i: (i, 0))
    agg = pl.BlockSpec((2, rb, 4), lambda i: (0, i, 0))

    stage_b = pl.pallas_call(
        _stage_b_body,
        grid=(grid,),
        in_specs=[pl.BlockSpec((2, rb), lambda i: (0, i)), row4],
        out_specs=[row4, row1, row1],
        out_shape=[
            jax.ShapeDtypeStruct((np_, 4), jnp.float32),
            jax.ShapeDtypeStruct((np_, 1), jnp.float32),
            jax.ShapeDtypeStruct((np_, 1), jnp.float32),
        ],
    )
    stage_d = pl.pallas_call(
        _stage_d_body,
        grid=(grid,),
        in_specs=[agg, row4, row1, row1, full((3, 16)), full((16,)),
                  full((16, 3))],
        out_specs=[row4, row3],
        out_shape=[
            jax.ShapeDtypeStruct((np_, 4), jnp.float32),
            jax.ShapeDtypeStruct((np_, 3), jnp.float32),
        ],
    )
    stage_f = pl.pallas_call(
        _stage_f_body,
        grid=(grid,),
        in_specs=[agg, row3, row1, full((3,))],
        out_specs=row3,
        out_shape=jax.ShapeDtypeStruct((np_, 3), jnp.float32),
    )
    return stage_b, stage_d, stage_f


def kernel(x, edge_index, W1, b1, W2, b2):
    n = x.shape[0]
    e = edge_index.shape[1]
    np_ = _node_pad(n)
    rp = _edge_rows_pad(e)
    tile_rows = np_ // NS

    src = edge_index[0]
    dst = edge_index[1]
    pad = rp * LN - e
    src1d = jnp.concatenate([src, jnp.zeros((pad,), jnp.int32)])
    dst1d = jnp.concatenate([dst, jnp.full((pad,), n, jnp.int32)])
    xp = jnp.pad(x, ((0, np_ - n), (0, 1)))

    zeros1 = jnp.zeros((tile_rows,), jnp.float32)
    zeros2 = jnp.zeros((tile_rows * 2,), jnp.float32)
    ones_l = jnp.ones((LN,), jnp.float32)

    deg_call = _make_deg_call(np_, rp)
    edge_call = _make_edge_call(np_, rp)
    stage_b, stage_d, stage_f = _dense_calls(np_)

    degp = deg_call(dst1d, zeros1, ones_l).reshape(NC, np_)
    vs1, dis, inv = stage_b(degp, xp)
    aggp1 = edge_call(vs1.reshape(-1), src1d, dst1d,
                      zeros2).reshape(NC, np_, 4)
    vs2, self2 = stage_d(aggp1, xp, dis, inv, W1, b1, W2)
    aggp2 = edge_call(vs2.reshape(-1), src1d, dst1d,
                      zeros2).reshape(NC, np_, 4)
    outp = stage_f(aggp2, self2, dis, b2)
    return outp[:n]
